# baseline (speedup 1.0000x reference)
import jax
import jax.numpy as jnp
from jax import lax
from jax.experimental import pallas as pl
from jax.experimental.pallas import tpu as pltpu
from jax.experimental.pallas import tpu_sc as plsc

VOCAB = 100000
HIDDEN = 128
BATCH = 1024
SEQ = 200
EPS = 1e-3

NC = 2
NS = 16
L = 16
NV = HIDDEN // L
NW = NC * NS
TOTAL = BATCH * SEQ
RW = TOTAL // NW
GRP = 128
G = RW // GRP
NBUF = 4
NCOMBO = 2 * SEQ


def _rsqrt(x):
    i = lax.bitcast_convert_type(x, jnp.int32)
    i = jnp.int32(0x5F3759DF) - lax.shift_right_arithmetic(i, jnp.int32(1))
    y = lax.bitcast_convert_type(i, jnp.float32)
    xh = x * 0.5
    for _ in range(2):
        y = y * (1.5 - xh * y * y)
    return y


def _body(tok_hbm, cidx_hbm, word_hbm, combo_hbm, gb_hbm, out_hbm,
          idx_v, cidx_v, wbuf, gb_v, combo_sh, sem_in, sem_add, sem_out):
    sid = lax.axis_index("s")
    wid = sid * NC + lax.axis_index("c")

    @pl.when(sid == 0)
    def _():
        pltpu.sync_copy(combo_hbm, combo_sh)
    plsc.subcore_barrier()

    pltpu.sync_copy(tok_hbm.at[wid], idx_v)
    pltpu.sync_copy(cidx_hbm.at[wid], cidx_v)
    pltpu.sync_copy(gb_hbm, gb_v)

    gammas = [gb_v[0, pl.ds(j * L, L)] for j in range(NV)]
    betas = [gb_v[1, pl.ds(j * L, L)] for j in range(NV)]
    inv_h = jnp.float32(1.0 / HIDDEN)
    iota16 = lax.iota(jnp.int32, L)
    iota8 = iota16 * 8

    def slot(i):
        return wbuf.at[pl.ds(lax.rem(i, NBUF) * GRP, GRP)]

    def drain(sem):
        pltpu.make_async_copy(out_hbm.at[pl.ds(0, GRP)],
                              wbuf.at[pl.ds(0, GRP)], sem).wait()

    pltpu.async_copy(word_hbm.at[idx_v.at[0]], slot(0), sem_in)
    drain(sem_in)
    pltpu.async_copy(combo_sh.at[cidx_v.at[0]], slot(0), sem_add, add=True)
    pltpu.async_copy(word_hbm.at[idx_v.at[1]], slot(1), sem_in)

    def chunk_body(c, _):
        r = lax.rem(c, NBUF)

        @pl.when(jnp.logical_and(c + 2 < G, c >= 2))
        def _():
            drain(sem_out)

        @pl.when(c + 2 < G)
        def _():
            pltpu.async_copy(word_hbm.at[idx_v.at[c + 2]], slot(c + 2),
                             sem_in)

        @pl.when(c + 1 < G)
        def _():
            drain(sem_in)
            pltpu.async_copy(combo_sh.at[cidx_v.at[c + 1]], slot(c + 1),
                             sem_add, add=True)

        drain(sem_add)

        rowbase = r * GRP
        bref = wbuf.at[pl.ds(rowbase, GRP)]

        def blk_body(blk, _):
            rows = lax.broadcast(rowbase + blk * L, (L,)) + iota16
            s = jnp.zeros((L,), jnp.float32)
            q = jnp.zeros((L,), jnp.float32)
            for h in range(HIDDEN):
                cols = lax.bitwise_and(iota8 + h, jnp.int32(HIDDEN - 1))
                x = plsc.load_gather(wbuf, [rows, cols])
                s = s + x
                q = x * x + q
            mean_v = s * inv_h
            var_v = q * inv_h - mean_v * mean_v
            rs_v = _rsqrt(var_v + EPS)
            blkref = bref.at[pl.ds(blk * L, L)]
            for k in range(L):
                mk = lax.broadcast(mean_v[k], (L,))
                rk = lax.broadcast(rs_v[k], (L,))
                for j in range(NV):
                    a = rk * gammas[j]
                    t = betas[j] - mk * a
                    blkref[k, pl.ds(j * L, L)] = \
                        blkref[k, pl.ds(j * L, L)] * a + t
            return ()

        lax.fori_loop(0, GRP // L, blk_body, ())

        base = wid * RW + c * GRP
        pltpu.async_copy(bref, out_hbm.at[pl.ds(base, GRP)], sem_out)
        return ()

    lax.fori_loop(0, G, chunk_body, ())
    for _ in range(NBUF):
        drain(sem_out)


@jax.jit
def _run(tok3, cidx3, word_emb, combo, gb):
    mesh = plsc.VectorSubcoreMesh(core_axis_name="c", subcore_axis_name="s",
                                  num_cores=NC, num_subcores=NS)
    f = pl.kernel(
        _body,
        out_type=jax.ShapeDtypeStruct((TOTAL, HIDDEN), jnp.float32),
        mesh=mesh,
        scratch_types=[
            pltpu.VMEM((G, GRP), jnp.int32),
            pltpu.VMEM((G, GRP), jnp.int32),
            pltpu.VMEM((NBUF * GRP, HIDDEN), jnp.float32),
            pltpu.VMEM((2, HIDDEN), jnp.float32),
            pltpu.VMEM_SHARED((NCOMBO, HIDDEN), jnp.float32),
            pltpu.SemaphoreType.DMA,
            pltpu.SemaphoreType.DMA,
            pltpu.SemaphoreType.DMA,
        ],
        compiler_params=pltpu.CompilerParams(needs_layout_passes=False),
    )
    return f(tok3, cidx3, word_emb, combo, gb)


def kernel(token, segment, word_emb, seg_emb, pos_emb, gamma, beta):
    tok3 = token.astype(jnp.int32).reshape(NW, G, GRP)
    pos = jnp.arange(SEQ, dtype=jnp.int32)
    cidx3 = (segment.astype(jnp.int32) * SEQ + pos[None, :]).reshape(NW, G, GRP)
    combo = (seg_emb[:, None, :] + pos_emb[None, :SEQ, :]).reshape(
        NCOMBO, HIDDEN)
    gb = jnp.stack([gamma, beta])
    out = _run(tok3, cidx3, word_emb, combo, gb)
    return out.reshape(BATCH, SEQ, HIDDEN)


# speedup vs baseline: 1.1063x; 1.1063x over previous
import jax
import jax.numpy as jnp
from jax import lax
from jax.experimental import pallas as pl
from jax.experimental.pallas import tpu as pltpu
from jax.experimental.pallas import tpu_sc as plsc

VOCAB = 100000
HIDDEN = 128
BATCH = 1024
SEQ = 200
EPS = 1e-3

NC = 2
NS = 16
L = 16
NV = HIDDEN // L
NW = NC * NS
TOTAL = BATCH * SEQ
RW = TOTAL // NW
GRP = 128
G = RW // GRP
NBUF = 4
NCOMBO = 2 * SEQ


def _rsqrt(x):
    i = lax.bitcast_convert_type(x, jnp.int32)
    i = jnp.int32(0x5F3759DF) - lax.shift_right_arithmetic(i, jnp.int32(1))
    y = lax.bitcast_convert_type(i, jnp.float32)
    xh = x * 0.5
    for _ in range(2):
        y = y * (1.5 - xh * y * y)
    return y


def _body(tok_hbm, cidx_hbm, word_hbm, combo_hbm, gb_hbm, out_hbm,
          idx_v, cidx_v, wbuf, gb_v, smat, combo_sh, sem_in, sem_add,
          sem_out):
    sid = lax.axis_index("s")
    wid = sid * NC + lax.axis_index("c")

    @pl.when(sid == 0)
    def _():
        pltpu.sync_copy(combo_hbm, combo_sh)
    plsc.subcore_barrier()

    pltpu.sync_copy(tok_hbm.at[wid], idx_v)
    pltpu.sync_copy(cidx_hbm.at[wid], cidx_v)
    pltpu.sync_copy(gb_hbm, gb_v)

    gammas = [gb_v[0, pl.ds(j * L, L)] for j in range(NV)]
    betas = [gb_v[1, pl.ds(j * L, L)] for j in range(NV)]
    inv_h = jnp.float32(1.0 / HIDDEN)
    iota16 = lax.iota(jnp.int32, L)

    def slot(i):
        return wbuf.at[pl.ds(lax.rem(i, NBUF) * GRP, GRP)]

    def drain(sem):
        pltpu.make_async_copy(out_hbm.at[pl.ds(0, GRP)],
                              wbuf.at[pl.ds(0, GRP)], sem).wait()

    pltpu.async_copy(word_hbm.at[idx_v.at[0]], slot(0), sem_in)
    drain(sem_in)
    pltpu.async_copy(combo_sh.at[cidx_v.at[0]], slot(0), sem_add, add=True)
    pltpu.async_copy(word_hbm.at[idx_v.at[1]], slot(1), sem_in)

    def chunk_body(c, _):
        r = lax.rem(c, NBUF)

        @pl.when(jnp.logical_and(c + 2 < G, c >= 2))
        def _():
            drain(sem_out)

        @pl.when(c + 2 < G)
        def _():
            pltpu.async_copy(word_hbm.at[idx_v.at[c + 2]], slot(c + 2),
                             sem_in)

        @pl.when(c + 1 < G)
        def _():
            drain(sem_in)
            pltpu.async_copy(combo_sh.at[cidx_v.at[c + 1]], slot(c + 1),
                             sem_add, add=True)

        drain(sem_add)

        bref = wbuf.at[pl.ds(r * GRP, GRP)]

        def blk_body(blk, _):
            blkref = bref.at[pl.ds(blk * L, L)]
            for k in range(L):
                xs = [blkref[k, pl.ds(j * L, L)] for j in range(NV)]
                s = ((xs[0] + xs[1]) + (xs[2] + xs[3])) + \
                    ((xs[4] + xs[5]) + (xs[6] + xs[7]))
                qs = [x * x for x in xs]
                q = ((qs[0] + qs[1]) + (qs[2] + qs[3])) + \
                    ((qs[4] + qs[5]) + (qs[6] + qs[7]))
                smat[k, pl.ds(0, L)] = s
                smat[L + k, pl.ds(0, L)] = q
            S = plsc.load_gather(smat, [iota16, jnp.zeros((L,), jnp.int32)])
            Q = plsc.load_gather(smat, [iota16 + L,
                                        jnp.zeros((L,), jnp.int32)])
            for l in range(1, L):
                cl = jnp.full((L,), l, jnp.int32)
                S = S + plsc.load_gather(smat, [iota16, cl])
                Q = Q + plsc.load_gather(smat, [iota16 + L, cl])
            mean_v = S * inv_h
            var_v = Q * inv_h - mean_v * mean_v
            rs_v = _rsqrt(var_v + EPS)
            for k in range(L):
                mk = lax.broadcast(mean_v[k], (L,))
                rk = lax.broadcast(rs_v[k], (L,))
                for j in range(NV):
                    a = rk * gammas[j]
                    t = betas[j] - mk * a
                    blkref[k, pl.ds(j * L, L)] = \
                        blkref[k, pl.ds(j * L, L)] * a + t
            return ()

        lax.fori_loop(0, GRP // L, blk_body, ())

        base = wid * RW + c * GRP
        pltpu.async_copy(bref, out_hbm.at[pl.ds(base, GRP)], sem_out)
        return ()

    lax.fori_loop(0, G, chunk_body, ())
    for _ in range(NBUF):
        drain(sem_out)


@jax.jit
def _run(tok3, cidx3, word_emb, combo, gb):
    mesh = plsc.VectorSubcoreMesh(core_axis_name="c", subcore_axis_name="s",
                                  num_cores=NC, num_subcores=NS)
    f = pl.kernel(
        _body,
        out_type=jax.ShapeDtypeStruct((TOTAL, HIDDEN), jnp.float32),
        mesh=mesh,
        scratch_types=[
            pltpu.VMEM((G, GRP), jnp.int32),
            pltpu.VMEM((G, GRP), jnp.int32),
            pltpu.VMEM((NBUF * GRP, HIDDEN), jnp.float32),
            pltpu.VMEM((2, HIDDEN), jnp.float32),
            pltpu.VMEM((2 * L, L + 1), jnp.float32),
            pltpu.VMEM_SHARED((NCOMBO, HIDDEN), jnp.float32),
            pltpu.SemaphoreType.DMA,
            pltpu.SemaphoreType.DMA,
            pltpu.SemaphoreType.DMA,
        ],
        compiler_params=pltpu.CompilerParams(needs_layout_passes=False),
    )
    return f(tok3, cidx3, word_emb, combo, gb)


def kernel(token, segment, word_emb, seg_emb, pos_emb, gamma, beta):
    tok3 = token.astype(jnp.int32).reshape(NW, G, GRP)
    pos = jnp.arange(SEQ, dtype=jnp.int32)
    cidx3 = (segment.astype(jnp.int32) * SEQ + pos[None, :]).reshape(NW, G, GRP)
    combo = (seg_emb[:, None, :] + pos_emb[None, :SEQ, :]).reshape(
        NCOMBO, HIDDEN)
    gb = jnp.stack([gamma, beta])
    out = _run(tok3, cidx3, word_emb, combo, gb)
    return out.reshape(BATCH, SEQ, HIDDEN)


# speedup vs baseline: 1.8490x; 1.6714x over previous
import jax
import jax.numpy as jnp
from jax import lax
from jax.experimental import pallas as pl
from jax.experimental.pallas import tpu as pltpu
from jax.experimental.pallas import tpu_sc as plsc

VOCAB = 100000
HIDDEN = 128
BATCH = 1024
SEQ = 200
EPS = 1e-3

NC = 2
NS = 16
L = 16
NV = HIDDEN // L
NW = NC * NS
TOTAL = BATCH * SEQ
RW = TOTAL // NW
GRP = 128
G = RW // GRP
NBUF = 4
NCOMBO = 2 * SEQ


def _rsqrt(x):
    i = lax.bitcast_convert_type(x, jnp.int32)
    i = jnp.int32(0x5F3759DF) - lax.shift_right_arithmetic(i, jnp.int32(1))
    y = lax.bitcast_convert_type(i, jnp.float32)
    xh = x * 0.5
    for _ in range(2):
        y = y * (1.5 - xh * y * y)
    return y


def _body(tok_hbm, cidx_hbm, word_hbm, combo_hbm, gb_hbm, out_hbm,
          idx_v, cidx_v, wbuf, gb_v, smat, combo_sh, sem_in, sem_add,
          sem_out):
    sid = lax.axis_index("s")
    wid = sid * NC + lax.axis_index("c")

    @pl.when(sid == 0)
    def _():
        pltpu.sync_copy(combo_hbm, combo_sh)
    plsc.subcore_barrier()

    pltpu.sync_copy(tok_hbm.at[wid], idx_v)
    pltpu.sync_copy(cidx_hbm.at[wid], cidx_v)
    pltpu.sync_copy(gb_hbm, gb_v)

    gammas = [gb_v[0, pl.ds(j * L, L)] for j in range(NV)]
    betas = [gb_v[1, pl.ds(j * L, L)] for j in range(NV)]
    inv_h = jnp.float32(1.0 / HIDDEN)
    iota16 = lax.iota(jnp.int32, L)

    def slot(i):
        return wbuf.at[pl.ds(lax.rem(i, NBUF) * GRP, GRP)]

    def drain(sem):
        pltpu.make_async_copy(out_hbm.at[pl.ds(0, GRP)],
                              wbuf.at[pl.ds(0, GRP)], sem).wait()

    pltpu.async_copy(word_hbm.at[idx_v.at[0]], slot(0), sem_in)
    drain(sem_in)
    pltpu.async_copy(combo_sh.at[cidx_v.at[0]], slot(0), sem_add, add=True)
    pltpu.async_copy(word_hbm.at[idx_v.at[1]], slot(1), sem_in)

    def chunk_body(c, _):
        r = lax.rem(c, NBUF)

        @pl.when(jnp.logical_and(c + 2 < G, c >= 2))
        def _():
            drain(sem_out)

        @pl.when(c + 2 < G)
        def _():
            pltpu.async_copy(word_hbm.at[idx_v.at[c + 2]], slot(c + 2),
                             sem_in)

        @pl.when(c + 1 < G)
        def _():
            drain(sem_in)
            pltpu.async_copy(combo_sh.at[cidx_v.at[c + 1]], slot(c + 1),
                             sem_add, add=True)

        drain(sem_add)

        bref = wbuf.at[pl.ds(r * GRP, GRP)]

        def blk_body(blk, _):
            blkref = bref.at[pl.ds(blk * L, L)]
            for k in range(L):
                xs = [blkref[k, pl.ds(j * L, L)] for j in range(NV)]
                s = ((xs[0] + xs[1]) + (xs[2] + xs[3])) + \
                    ((xs[4] + xs[5]) + (xs[6] + xs[7]))
                mean = lax.broadcast(jnp.sum(s), (L,)) * inv_h
                qs = [x * x for x in xs]
                q = ((qs[0] + qs[1]) + (qs[2] + qs[3])) + \
                    ((qs[4] + qs[5]) + (qs[6] + qs[7]))
                ex2 = lax.broadcast(jnp.sum(q), (L,)) * inv_h
                rs = _rsqrt(ex2 - mean * mean + EPS)
                for j in range(NV):
                    a = rs * gammas[j]
                    t = betas[j] - mean * a
                    blkref[k, pl.ds(j * L, L)] = xs[j] * a + t
            return ()

        lax.fori_loop(0, GRP // L, blk_body, ())

        base = wid * RW + c * GRP
        pltpu.async_copy(bref, out_hbm.at[pl.ds(base, GRP)], sem_out)
        return ()

    lax.fori_loop(0, G, chunk_body, ())
    for _ in range(NBUF):
        drain(sem_out)


@jax.jit
def _run(tok3, cidx3, word_emb, combo, gb):
    mesh = plsc.VectorSubcoreMesh(core_axis_name="c", subcore_axis_name="s",
                                  num_cores=NC, num_subcores=NS)
    f = pl.kernel(
        _body,
        out_type=jax.ShapeDtypeStruct((TOTAL, HIDDEN), jnp.float32),
        mesh=mesh,
        scratch_types=[
            pltpu.VMEM((G, GRP), jnp.int32),
            pltpu.VMEM((G, GRP), jnp.int32),
            pltpu.VMEM((NBUF * GRP, HIDDEN), jnp.float32),
            pltpu.VMEM((2, HIDDEN), jnp.float32),
            pltpu.VMEM((2 * L, L + 1), jnp.float32),
            pltpu.VMEM_SHARED((NCOMBO, HIDDEN), jnp.float32),
            pltpu.SemaphoreType.DMA,
            pltpu.SemaphoreType.DMA,
            pltpu.SemaphoreType.DMA,
        ],
        compiler_params=pltpu.CompilerParams(needs_layout_passes=False),
    )
    return f(tok3, cidx3, word_emb, combo, gb)


def kernel(token, segment, word_emb, seg_emb, pos_emb, gamma, beta):
    tok3 = token.astype(jnp.int32).reshape(NW, G, GRP)
    pos = jnp.arange(SEQ, dtype=jnp.int32)
    cidx3 = (segment.astype(jnp.int32) * SEQ + pos[None, :]).reshape(NW, G, GRP)
    combo = (seg_emb[:, None, :] + pos_emb[None, :SEQ, :]).reshape(
        NCOMBO, HIDDEN)
    gb = jnp.stack([gamma, beta])
    out = _run(tok3, cidx3, word_emb, combo, gb)
    return out.reshape(BATCH, SEQ, HIDDEN)


# speedup vs baseline: 2.5183x; 1.3620x over previous
import jax
import jax.numpy as jnp
from jax import lax
from jax.experimental import pallas as pl
from jax.experimental.pallas import tpu as pltpu
from jax.experimental.pallas import tpu_sc as plsc

VOCAB = 100000
HIDDEN = 128
BATCH = 1024
SEQ = 200
EPS = 1e-3

NC = 2
NS = 16
L = 16
NW = NC * NS
TOTAL = BATCH * SEQ
RW = TOTAL // NW
GRP = 128
G = RW // GRP
NBUF = 4
NCOMBO = 2 * SEQ
RB = 2048


def _gather_body(tok_hbm, cidx_hbm, word_hbm, combo_hbm, out_hbm,
                 idx_v, cidx_v, wbuf, combo_sh, sem_in, sem_add, sem_out):
    sid = lax.axis_index("s")
    wid = sid * NC + lax.axis_index("c")

    @pl.when(sid == 0)
    def _():
        pltpu.sync_copy(combo_hbm, combo_sh)
    plsc.subcore_barrier()

    pltpu.sync_copy(tok_hbm.at[wid], idx_v)
    pltpu.sync_copy(cidx_hbm.at[wid], cidx_v)

    def slot(i):
        return wbuf.at[pl.ds(lax.rem(i, NBUF) * GRP, GRP)]

    def drain(sem):
        pltpu.make_async_copy(out_hbm.at[pl.ds(0, GRP)],
                              wbuf.at[pl.ds(0, GRP)], sem).wait()

    pltpu.async_copy(word_hbm.at[idx_v.at[0]], slot(0), sem_in)
    drain(sem_in)
    pltpu.async_copy(combo_sh.at[cidx_v.at[0]], slot(0), sem_add, add=True)
    pltpu.async_copy(word_hbm.at[idx_v.at[1]], slot(1), sem_in)

    def chunk_body(c, _):
        @pl.when(jnp.logical_and(c + 2 < G, c >= 2))
        def _():
            drain(sem_out)

        @pl.when(c + 2 < G)
        def _():
            pltpu.async_copy(word_hbm.at[idx_v.at[c + 2]], slot(c + 2),
                             sem_in)

        @pl.when(c + 1 < G)
        def _():
            drain(sem_in)
            pltpu.async_copy(combo_sh.at[cidx_v.at[c + 1]], slot(c + 1),
                             sem_add, add=True)

        drain(sem_add)
        base = wid * RW + c * GRP
        pltpu.async_copy(slot(c), out_hbm.at[pl.ds(base, GRP)], sem_out)
        return ()

    lax.fori_loop(0, G, chunk_body, ())
    for _ in range(NBUF):
        drain(sem_out)


def _ln_body(x_ref, gb_ref, o_ref):
    x = x_ref[...]
    mean = jnp.mean(x, axis=1, keepdims=True)
    xc = x - mean
    var = jnp.mean(xc * xc, axis=1, keepdims=True)
    o_ref[...] = (xc * lax.rsqrt(var + EPS) * gb_ref[0:1, :]
                  + gb_ref[1:2, :])


@jax.jit
def _run(tok3, cidx3, word_emb, combo, gb):
    mesh = plsc.VectorSubcoreMesh(core_axis_name="c", subcore_axis_name="s",
                                  num_cores=NC, num_subcores=NS)
    gather = pl.kernel(
        _gather_body,
        out_type=jax.ShapeDtypeStruct((TOTAL, HIDDEN), jnp.float32),
        mesh=mesh,
        scratch_types=[
            pltpu.VMEM((G, GRP), jnp.int32),
            pltpu.VMEM((G, GRP), jnp.int32),
            pltpu.VMEM((NBUF * GRP, HIDDEN), jnp.float32),
            pltpu.VMEM_SHARED((NCOMBO, HIDDEN), jnp.float32),
            pltpu.SemaphoreType.DMA,
            pltpu.SemaphoreType.DMA,
            pltpu.SemaphoreType.DMA,
        ],
        compiler_params=pltpu.CompilerParams(needs_layout_passes=False),
    )
    x = gather(tok3, cidx3, word_emb, combo)
    ln = pl.pallas_call(
        _ln_body,
        grid=(TOTAL // RB,),
        in_specs=[pl.BlockSpec((RB, HIDDEN), lambda i: (i, 0)),
                  pl.BlockSpec((2, HIDDEN), lambda i: (0, 0))],
        out_specs=pl.BlockSpec((RB, HIDDEN), lambda i: (i, 0)),
        out_shape=jax.ShapeDtypeStruct((TOTAL, HIDDEN), jnp.float32),
    )
    return ln(x, gb)


def kernel(token, segment, word_emb, seg_emb, pos_emb, gamma, beta):
    tok3 = token.astype(jnp.int32).reshape(NW, G, GRP)
    pos = jnp.arange(SEQ, dtype=jnp.int32)
    cidx3 = (segment.astype(jnp.int32) * SEQ + pos[None, :]).reshape(NW, G, GRP)
    combo = (seg_emb[:, None, :] + pos_emb[None, :SEQ, :]).reshape(
        NCOMBO, HIDDEN)
    gb = jnp.stack([gamma, beta])
    out = _run(tok3, cidx3, word_emb, combo, gb)
    return out.reshape(BATCH, SEQ, HIDDEN)


# speedup vs baseline: 2.5346x; 1.0065x over previous
import jax
import jax.numpy as jnp
from jax import lax
from jax.experimental import pallas as pl
from jax.experimental.pallas import tpu as pltpu
from jax.experimental.pallas import tpu_sc as plsc

VOCAB = 100000
HIDDEN = 128
BATCH = 1024
SEQ = 200
EPS = 1e-3

NC = 2
NS = 16
L = 16
NW = NC * NS
TOTAL = BATCH * SEQ
RW = TOTAL // NW
GRP = 128
G = RW // GRP
NBUF = 4
NCOMBO = 2 * SEQ
RB = 2048


def _gather_body(tok_hbm, cidx_hbm, word_hbm, combo_hbm, out_hbm,
                 idx_v, cidx_v, wbuf, combo_sh, sem_in, sem_add, sem_out):
    sid = lax.axis_index("s")
    wid = sid * NC + lax.axis_index("c")

    @pl.when(sid == 0)
    def _():
        pltpu.sync_copy(combo_hbm, combo_sh)
    plsc.subcore_barrier()

    pltpu.sync_copy(tok_hbm.at[wid], idx_v)
    pltpu.sync_copy(cidx_hbm.at[wid], cidx_v)

    def slot(i):
        return wbuf.at[pl.ds(lax.rem(i, NBUF) * GRP, GRP)]

    def drain(sem):
        pltpu.make_async_copy(out_hbm.at[pl.ds(0, GRP)],
                              wbuf.at[pl.ds(0, GRP)], sem).wait()

    pltpu.async_copy(word_hbm.at[idx_v.at[0]], slot(0), sem_in)
    drain(sem_in)
    pltpu.async_copy(combo_sh.at[cidx_v.at[0]], slot(0), sem_add, add=True)
    pltpu.async_copy(word_hbm.at[idx_v.at[1]], slot(1), sem_in)

    def chunk_body(c, _):
        @pl.when(jnp.logical_and(c + 2 < G, c >= 2))
        def _():
            drain(sem_out)

        @pl.when(c + 2 < G)
        def _():
            pltpu.async_copy(word_hbm.at[idx_v.at[c + 2]], slot(c + 2),
                             sem_in)

        @pl.when(c + 1 < G)
        def _():
            drain(sem_in)
            pltpu.async_copy(combo_sh.at[cidx_v.at[c + 1]], slot(c + 1),
                             sem_add, add=True)

        drain(sem_add)
        base = wid * RW + c * GRP
        pltpu.async_copy(slot(c), out_hbm.at[pl.ds(base, GRP)], sem_out)
        return ()

    lax.fori_loop(0, G, chunk_body, ())
    for _ in range(NBUF):
        drain(sem_out)


def _ln_body(x_ref, gb_ref, o_ref):
    x = x_ref[...]
    ones = jnp.ones((HIDDEN, HIDDEN), jnp.float32)
    mean = jax.lax.dot(x, ones) * (1.0 / HIDDEN)
    xc = x - mean
    var = jax.lax.dot(xc * xc, ones) * (1.0 / HIDDEN)
    o_ref[...] = (xc * lax.rsqrt(var + EPS) * gb_ref[0:1, :]
                  + gb_ref[1:2, :])


@jax.jit
def _run(tok3, cidx3, word_emb, combo, gb):
    mesh = plsc.VectorSubcoreMesh(core_axis_name="c", subcore_axis_name="s",
                                  num_cores=NC, num_subcores=NS)
    gather = pl.kernel(
        _gather_body,
        out_type=jax.ShapeDtypeStruct((TOTAL, HIDDEN), jnp.float32),
        mesh=mesh,
        scratch_types=[
            pltpu.VMEM((G, GRP), jnp.int32),
            pltpu.VMEM((G, GRP), jnp.int32),
            pltpu.VMEM((NBUF * GRP, HIDDEN), jnp.float32),
            pltpu.VMEM_SHARED((NCOMBO, HIDDEN), jnp.float32),
            pltpu.SemaphoreType.DMA,
            pltpu.SemaphoreType.DMA,
            pltpu.SemaphoreType.DMA,
        ],
        compiler_params=pltpu.CompilerParams(needs_layout_passes=False),
    )
    x = gather(tok3, cidx3, word_emb, combo)
    ln = pl.pallas_call(
        _ln_body,
        grid=(TOTAL // RB,),
        in_specs=[pl.BlockSpec((RB, HIDDEN), lambda i: (i, 0)),
                  pl.BlockSpec((2, HIDDEN), lambda i: (0, 0))],
        out_specs=pl.BlockSpec((RB, HIDDEN), lambda i: (i, 0)),
        out_shape=jax.ShapeDtypeStruct((TOTAL, HIDDEN), jnp.float32),
    )
    return ln(x, gb)


def kernel(token, segment, word_emb, seg_emb, pos_emb, gamma, beta):
    tok3 = token.astype(jnp.int32).reshape(NW, G, GRP)
    pos = jnp.arange(SEQ, dtype=jnp.int32)
    cidx3 = (segment.astype(jnp.int32) * SEQ + pos[None, :]).reshape(NW, G, GRP)
    combo = (seg_emb[:, None, :] + pos_emb[None, :SEQ, :]).reshape(
        NCOMBO, HIDDEN)
    gb = jnp.stack([gamma, beta])
    out = _run(tok3, cidx3, word_emb, combo, gb)
    return out.reshape(BATCH, SEQ, HIDDEN)


# speedup vs baseline: 2.6894x; 1.0611x over previous
import jax
import jax.numpy as jnp
from jax import lax
from jax.experimental import pallas as pl
from jax.experimental.pallas import tpu as pltpu
from jax.experimental.pallas import tpu_sc as plsc

VOCAB = 100000
HIDDEN = 128
BATCH = 1024
SEQ = 200
EPS = 1e-3

NC = 2
NS = 16
L = 16
NW = NC * NS
TOTAL = BATCH * SEQ
RW = TOTAL // NW
GRP = 128
G = RW // GRP
NBUF = 4
NCOMBO = 2 * SEQ
RB = 2048
NSLICE = 2
GS = G // NSLICE
RWS = GS * GRP
ROWS_S = TOTAL // NSLICE


def _gather_body(tok_hbm, cidx_hbm, word_hbm, combo_hbm, out_hbm,
                 idx_v, cidx_v, wbuf, combo_sh, sem_in, sem_add, sem_out):
    sid = lax.axis_index("s")
    wid = sid * NC + lax.axis_index("c")

    @pl.when(sid == 0)
    def _():
        pltpu.sync_copy(combo_hbm, combo_sh)
    plsc.subcore_barrier()

    pltpu.sync_copy(tok_hbm.at[wid], idx_v)
    pltpu.sync_copy(cidx_hbm.at[wid], cidx_v)

    def slot(i):
        return wbuf.at[pl.ds(lax.rem(i, NBUF) * GRP, GRP)]

    def drain(sem):
        pltpu.make_async_copy(out_hbm.at[pl.ds(0, GRP)],
                              wbuf.at[pl.ds(0, GRP)], sem).wait()

    pltpu.async_copy(word_hbm.at[idx_v.at[0]], slot(0), sem_in)
    drain(sem_in)
    pltpu.async_copy(combo_sh.at[cidx_v.at[0]], slot(0), sem_add, add=True)
    pltpu.async_copy(word_hbm.at[idx_v.at[1]], slot(1), sem_in)

    def chunk_body(c, _):
        @pl.when(jnp.logical_and(c + 2 < GS, c >= 2))
        def _():
            drain(sem_out)

        @pl.when(c + 2 < GS)
        def _():
            pltpu.async_copy(word_hbm.at[idx_v.at[c + 2]], slot(c + 2),
                             sem_in)

        @pl.when(c + 1 < GS)
        def _():
            drain(sem_in)
            pltpu.async_copy(combo_sh.at[cidx_v.at[c + 1]], slot(c + 1),
                             sem_add, add=True)

        drain(sem_add)
        base = wid * RWS + c * GRP
        pltpu.async_copy(slot(c), out_hbm.at[pl.ds(base, GRP)], sem_out)
        return ()

    lax.fori_loop(0, GS, chunk_body, ())
    for _ in range(NBUF):
        drain(sem_out)


def _ln_body(x_ref, gb_ref, o_ref):
    x = x_ref[...]
    ones = jnp.ones((HIDDEN, HIDDEN), jnp.float32)
    mean = jax.lax.dot(x, ones) * (1.0 / HIDDEN)
    xc = x - mean
    var = jax.lax.dot(xc * xc, ones) * (1.0 / HIDDEN)
    o_ref[...] = (xc * lax.rsqrt(var + EPS) * gb_ref[0:1, :]
                  + gb_ref[1:2, :])


@jax.jit
def _run(tok4, cidx4, word_emb, combo, gb):
    mesh = plsc.VectorSubcoreMesh(core_axis_name="c", subcore_axis_name="s",
                                  num_cores=NC, num_subcores=NS)
    gather = pl.kernel(
        _gather_body,
        out_type=jax.ShapeDtypeStruct((ROWS_S, HIDDEN), jnp.float32),
        mesh=mesh,
        scratch_types=[
            pltpu.VMEM((GS, GRP), jnp.int32),
            pltpu.VMEM((GS, GRP), jnp.int32),
            pltpu.VMEM((NBUF * GRP, HIDDEN), jnp.float32),
            pltpu.VMEM_SHARED((NCOMBO, HIDDEN), jnp.float32),
            pltpu.SemaphoreType.DMA,
            pltpu.SemaphoreType.DMA,
            pltpu.SemaphoreType.DMA,
        ],
        compiler_params=pltpu.CompilerParams(needs_layout_passes=False),
    )

    nblk = ROWS_S // RB

    def make_ln(s, aliased):
        in_specs = [pl.BlockSpec((RB, HIDDEN), lambda i: (i, 0)),
                    pl.BlockSpec((2, HIDDEN), lambda i: (0, 0))]
        body = _ln_body
        io_aliases = {}
        if aliased:
            in_specs = [pl.BlockSpec(memory_space=pl.ANY)] + in_specs
            body = lambda buf_ref, x_ref, gb_ref, o_ref: \
                _ln_body(x_ref, gb_ref, o_ref)
            io_aliases = {0: 0}
        return pl.pallas_call(
            body,
            grid=(nblk,),
            in_specs=in_specs,
            out_specs=pl.BlockSpec((RB, HIDDEN),
                                   lambda i, s=s: (i + s * nblk, 0)),
            out_shape=jax.ShapeDtypeStruct((TOTAL, HIDDEN), jnp.float32),
            input_output_aliases=io_aliases,
        )

    xs = [gather(tok4[s], cidx4[s], word_emb, combo) for s in range(NSLICE)]
    out = make_ln(0, False)(xs[0], gb)
    for s in range(1, NSLICE):
        out = make_ln(s, True)(out, xs[s], gb)
    return out


def kernel(token, segment, word_emb, seg_emb, pos_emb, gamma, beta):
    tok4 = token.astype(jnp.int32).reshape(NSLICE, NW, GS, GRP)
    pos = jnp.arange(SEQ, dtype=jnp.int32)
    cidx4 = (segment.astype(jnp.int32) * SEQ + pos[None, :]).reshape(
        NSLICE, NW, GS, GRP)
    combo = (seg_emb[:, None, :] + pos_emb[None, :SEQ, :]).reshape(
        NCOMBO, HIDDEN)
    gb = jnp.stack([gamma, beta])
    out = _run(tok4, cidx4, word_emb, combo, gb)
    return out.reshape(BATCH, SEQ, HIDDEN)


# speedup vs baseline: 2.7587x; 1.0258x over previous
import jax
import jax.numpy as jnp
from jax import lax
from jax.experimental import pallas as pl
from jax.experimental.pallas import tpu as pltpu
from jax.experimental.pallas import tpu_sc as plsc

VOCAB = 100000
HIDDEN = 128
BATCH = 1024
SEQ = 200
EPS = 1e-3

NC = 2
NS = 16
L = 16
NW = NC * NS
TOTAL = BATCH * SEQ
RW = TOTAL // NW
GRP = 128
G = RW // GRP
NBUF = 4
NCOMBO = 2 * SEQ
RB = 2048
NSLICE = 5
GS = G // NSLICE
RWS = GS * GRP
ROWS_S = TOTAL // NSLICE


def _gather_body(tok_hbm, cidx_hbm, word_hbm, combo_hbm, out_hbm,
                 idx_v, cidx_v, wbuf, combo_sh, sem_in, sem_add, sem_out):
    sid = lax.axis_index("s")
    wid = sid * NC + lax.axis_index("c")

    @pl.when(sid == 0)
    def _():
        pltpu.sync_copy(combo_hbm, combo_sh)
    plsc.subcore_barrier()

    pltpu.sync_copy(tok_hbm.at[wid], idx_v)
    pltpu.sync_copy(cidx_hbm.at[wid], cidx_v)

    def slot(i):
        return wbuf.at[pl.ds(lax.rem(i, NBUF) * GRP, GRP)]

    def drain(sem):
        pltpu.make_async_copy(out_hbm.at[pl.ds(0, GRP)],
                              wbuf.at[pl.ds(0, GRP)], sem).wait()

    pltpu.async_copy(word_hbm.at[idx_v.at[0]], slot(0), sem_in)
    drain(sem_in)
    pltpu.async_copy(combo_sh.at[cidx_v.at[0]], slot(0), sem_add, add=True)
    pltpu.async_copy(word_hbm.at[idx_v.at[1]], slot(1), sem_in)

    def chunk_body(c, _):
        @pl.when(jnp.logical_and(c + 2 < GS, c >= 2))
        def _():
            drain(sem_out)

        @pl.when(c + 2 < GS)
        def _():
            pltpu.async_copy(word_hbm.at[idx_v.at[c + 2]], slot(c + 2),
                             sem_in)

        @pl.when(c + 1 < GS)
        def _():
            drain(sem_in)
            pltpu.async_copy(combo_sh.at[cidx_v.at[c + 1]], slot(c + 1),
                             sem_add, add=True)

        drain(sem_add)
        base = wid * RWS + c * GRP
        pltpu.async_copy(slot(c), out_hbm.at[pl.ds(base, GRP)], sem_out)
        return ()

    lax.fori_loop(0, GS, chunk_body, ())
    for _ in range(NBUF):
        drain(sem_out)


def _ln_body(x_ref, gb_ref, o_ref):
    x = x_ref[...]
    ones = jnp.ones((HIDDEN, HIDDEN), jnp.float32)
    mean = jax.lax.dot(x, ones) * (1.0 / HIDDEN)
    xc = x - mean
    var = jax.lax.dot(xc * xc, ones) * (1.0 / HIDDEN)
    o_ref[...] = (xc * lax.rsqrt(var + EPS) * gb_ref[0:1, :]
                  + gb_ref[1:2, :])


@jax.jit
def _run(tok4, cidx4, word_emb, combo, gb):
    mesh = plsc.VectorSubcoreMesh(core_axis_name="c", subcore_axis_name="s",
                                  num_cores=NC, num_subcores=NS)
    gather = pl.kernel(
        _gather_body,
        out_type=jax.ShapeDtypeStruct((ROWS_S, HIDDEN), jnp.float32),
        mesh=mesh,
        scratch_types=[
            pltpu.VMEM((GS, GRP), jnp.int32),
            pltpu.VMEM((GS, GRP), jnp.int32),
            pltpu.VMEM((NBUF * GRP, HIDDEN), jnp.float32),
            pltpu.VMEM_SHARED((NCOMBO, HIDDEN), jnp.float32),
            pltpu.SemaphoreType.DMA,
            pltpu.SemaphoreType.DMA,
            pltpu.SemaphoreType.DMA,
        ],
        compiler_params=pltpu.CompilerParams(needs_layout_passes=False),
    )

    nblk = ROWS_S // RB

    def make_ln(s, aliased):
        in_specs = [pl.BlockSpec((RB, HIDDEN), lambda i: (i, 0)),
                    pl.BlockSpec((2, HIDDEN), lambda i: (0, 0))]
        body = _ln_body
        io_aliases = {}
        if aliased:
            in_specs = [pl.BlockSpec(memory_space=pl.ANY)] + in_specs
            body = lambda buf_ref, x_ref, gb_ref, o_ref: \
                _ln_body(x_ref, gb_ref, o_ref)
            io_aliases = {0: 0}
        return pl.pallas_call(
            body,
            grid=(nblk,),
            in_specs=in_specs,
            out_specs=pl.BlockSpec((RB, HIDDEN),
                                   lambda i, s=s: (i + s * nblk, 0)),
            out_shape=jax.ShapeDtypeStruct((TOTAL, HIDDEN), jnp.float32),
            input_output_aliases=io_aliases,
        )

    xs = [gather(tok4[s], cidx4[s], word_emb, combo) for s in range(NSLICE)]
    out = make_ln(0, False)(xs[0], gb)
    for s in range(1, NSLICE):
        out = make_ln(s, True)(out, xs[s], gb)
    return out


def kernel(token, segment, word_emb, seg_emb, pos_emb, gamma, beta):
    tok4 = token.astype(jnp.int32).reshape(NSLICE, NW, GS, GRP)
    pos = jnp.arange(SEQ, dtype=jnp.int32)
    cidx4 = (segment.astype(jnp.int32) * SEQ + pos[None, :]).reshape(
        NSLICE, NW, GS, GRP)
    combo = (seg_emb[:, None, :] + pos_emb[None, :SEQ, :]).reshape(
        NCOMBO, HIDDEN)
    gb = jnp.stack([gamma, beta])
    out = _run(tok4, cidx4, word_emb, combo, gb)
    return out.reshape(BATCH, SEQ, HIDDEN)


# speedup vs baseline: 3.0295x; 1.0981x over previous
import jax
import jax.numpy as jnp
from jax import lax
from jax.experimental import pallas as pl
from jax.experimental.pallas import tpu as pltpu
from jax.experimental.pallas import tpu_sc as plsc

VOCAB = 100000
HIDDEN = 128
BATCH = 1024
SEQ = 200
EPS = 1e-3

NC = 2
NS = 16
L = 16
NW = NC * NS
TOTAL = BATCH * SEQ
RW = TOTAL // NW
GRP = 128
G = RW // GRP
NBUF = 4
NCOMBO = 2 * SEQ
RB = 4096
NSLICE = 5
GS = G // NSLICE
RWS = GS * GRP
ROWS_S = TOTAL // NSLICE


def _gather_body(tok_hbm, cidx_hbm, word_hbm, combo_hbm, out_hbm,
                 idx_v, cidx_v, wbuf, combo_sh, sem_in, sem_add, sem_out):
    sid = lax.axis_index("s")
    wid = sid * NC + lax.axis_index("c")

    @pl.when(sid == 0)
    def _():
        pltpu.sync_copy(combo_hbm, combo_sh)
    plsc.subcore_barrier()

    pltpu.sync_copy(tok_hbm.at[wid], idx_v)
    pltpu.sync_copy(cidx_hbm.at[wid], cidx_v)

    def slot(i):
        return wbuf.at[pl.ds(lax.rem(i, NBUF) * GRP, GRP)]

    def drain(sem):
        pltpu.make_async_copy(out_hbm.at[pl.ds(0, GRP)],
                              wbuf.at[pl.ds(0, GRP)], sem).wait()

    pltpu.async_copy(word_hbm.at[idx_v.at[0]], slot(0), sem_in)
    drain(sem_in)
    pltpu.async_copy(combo_sh.at[cidx_v.at[0]], slot(0), sem_add, add=True)
    pltpu.async_copy(word_hbm.at[idx_v.at[1]], slot(1), sem_in)

    def chunk_body(c, _):
        @pl.when(jnp.logical_and(c + 2 < GS, c >= 2))
        def _():
            drain(sem_out)

        @pl.when(c + 2 < GS)
        def _():
            pltpu.async_copy(word_hbm.at[idx_v.at[c + 2]], slot(c + 2),
                             sem_in)

        @pl.when(c + 1 < GS)
        def _():
            drain(sem_in)
            pltpu.async_copy(combo_sh.at[cidx_v.at[c + 1]], slot(c + 1),
                             sem_add, add=True)

        drain(sem_add)
        base = wid * RWS + c * GRP
        pltpu.async_copy(slot(c), out_hbm.at[pl.ds(base, GRP)], sem_out)
        return ()

    lax.fori_loop(0, GS, chunk_body, ())
    for _ in range(NBUF):
        drain(sem_out)


def _ln_body(x_ref, gb_ref, o_ref):
    x = x_ref[...]
    ones = jnp.ones((HIDDEN, HIDDEN), jnp.float32)
    mean = jax.lax.dot(x, ones) * (1.0 / HIDDEN)
    xc = x - mean
    var = jax.lax.dot(xc * xc, ones) * (1.0 / HIDDEN)
    o_ref[...] = (xc * lax.rsqrt(var + EPS) * gb_ref[0:1, :]
                  + gb_ref[1:2, :])


@jax.jit
def _run(tok4, cidx4, word_emb, combo, gb):
    mesh = plsc.VectorSubcoreMesh(core_axis_name="c", subcore_axis_name="s",
                                  num_cores=NC, num_subcores=NS)
    gather = pl.kernel(
        _gather_body,
        out_type=jax.ShapeDtypeStruct((ROWS_S, HIDDEN), jnp.float32),
        mesh=mesh,
        scratch_types=[
            pltpu.VMEM((GS, GRP), jnp.int32),
            pltpu.VMEM((GS, GRP), jnp.int32),
            pltpu.VMEM((NBUF * GRP, HIDDEN), jnp.float32),
            pltpu.VMEM_SHARED((NCOMBO, HIDDEN), jnp.float32),
            pltpu.SemaphoreType.DMA,
            pltpu.SemaphoreType.DMA,
            pltpu.SemaphoreType.DMA,
        ],
        compiler_params=pltpu.CompilerParams(needs_layout_passes=False),
    )

    nblk = ROWS_S // RB

    def make_ln(s, aliased):
        in_specs = [pl.BlockSpec((RB, HIDDEN), lambda i: (i, 0)),
                    pl.BlockSpec((2, HIDDEN), lambda i: (0, 0))]
        body = _ln_body
        io_aliases = {}
        if aliased:
            in_specs = [pl.BlockSpec(memory_space=pl.ANY)] + in_specs
            body = lambda buf_ref, x_ref, gb_ref, o_ref: \
                _ln_body(x_ref, gb_ref, o_ref)
            io_aliases = {0: 0}
        return pl.pallas_call(
            body,
            grid=(nblk,),
            in_specs=in_specs,
            out_specs=pl.BlockSpec((RB, HIDDEN),
                                   lambda i, s=s: (i + s * nblk, 0)),
            out_shape=jax.ShapeDtypeStruct((TOTAL, HIDDEN), jnp.float32),
            input_output_aliases=io_aliases,
        )

    xs = [gather(tok4[s], cidx4[s], word_emb, combo) for s in range(NSLICE)]
    out = make_ln(0, False)(xs[0], gb)
    for s in range(1, NSLICE):
        out = make_ln(s, True)(out, xs[s], gb)
    return out


def kernel(token, segment, word_emb, seg_emb, pos_emb, gamma, beta):
    tok4 = token.astype(jnp.int32).reshape(NSLICE, NW, GS, GRP)
    pos = jnp.arange(SEQ, dtype=jnp.int32)
    cidx4 = (segment.astype(jnp.int32) * SEQ + pos[None, :]).reshape(
        NSLICE, NW, GS, GRP)
    combo = (seg_emb[:, None, :] + pos_emb[None, :SEQ, :]).reshape(
        NCOMBO, HIDDEN)
    gb = jnp.stack([gamma, beta])
    out = _run(tok4, cidx4, word_emb, combo, gb)
    return out.reshape(BATCH, SEQ, HIDDEN)
